# Initial kernel scaffold; baseline (speedup 1.0000x reference)
import jax
import jax.numpy as jnp
from jax.experimental import pallas as pl


def kernel(features, edge_index, betas, W):
    raise NotImplementedError("write your pallas kernel here")


# speedup vs baseline: 1.9849x; 1.9849x over previous
import functools

import jax
import jax.numpy as jnp
from jax import lax
from jax.experimental import pallas as pl
from jax.experimental.pallas import tpu as pltpu
from jax.experimental.pallas import tpu_sc as plsc

N = 10000
NP = 10240
D = 128
E = 320000
C = 64
NWORK = 32
EPW = E // NWORK
CB = 80
NCH = EPW // CB
RPT = NP // 16
EPS = 1e-12


def _prep_body(x_ref, b_ref, hn_ref, hb_ref, nrm_ref):
    x = x_ref[...]
    nrm = jnp.sqrt(jnp.sum(x * x, axis=1, keepdims=True))
    hn = x / jnp.maximum(nrm, EPS)
    hn_ref[...] = hn
    hb_ref[...] = hn * b_ref[0]
    nrm_ref[...] = nrm[:, 0]


def _prep(xpad, beta):
    return pl.pallas_call(
        _prep_body,
        out_shape=[
            jax.ShapeDtypeStruct((NP, D), jnp.float32),
            jax.ShapeDtypeStruct((NP, D), jnp.float32),
            jax.ShapeDtypeStruct((NP,), jnp.float32),
        ],
        in_specs=[
            pl.BlockSpec(memory_space=pltpu.VMEM),
            pl.BlockSpec(memory_space=pltpu.SMEM),
        ],
    )(xpad, beta)


def _combine_body(u_ref, den_ref, b_ref, hn_ref, hb_ref, nrm_ref):
    u = u_ref[0] + u_ref[1]
    den = den_ref[0] + den_ref[1]
    den = jnp.where(den == 0.0, 1.0, den)
    x = u / den[:, None]
    nrm = jnp.sqrt(jnp.sum(x * x, axis=1, keepdims=True))
    hn = x / jnp.maximum(nrm, EPS)
    hn_ref[...] = hn
    hb_ref[...] = hn * b_ref[0]
    nrm_ref[...] = nrm[:, 0]


def _combine(u, den, beta):
    return pl.pallas_call(
        _combine_body,
        out_shape=[
            jax.ShapeDtypeStruct((NP, D), jnp.float32),
            jax.ShapeDtypeStruct((NP, D), jnp.float32),
            jax.ShapeDtypeStruct((NP,), jnp.float32),
        ],
        in_specs=[
            pl.BlockSpec(memory_space=pltpu.VMEM),
            pl.BlockSpec(memory_space=pltpu.VMEM),
            pl.BlockSpec(memory_space=pltpu.SMEM),
        ],
    )(u, den, beta)


def _final_body(u_ref, den_ref, w_ref, out_ref):
    u = u_ref[0] + u_ref[1]
    den = den_ref[0] + den_ref[1]
    den = jnp.where(den == 0.0, 1.0, den)
    x = u / den[:, None]
    out_ref[...] = lax.dot_general(
        x, w_ref[...], (((1,), (1,)), ((), ())),
        preferred_element_type=jnp.float32)


def _final(u, den, W):
    return pl.pallas_call(
        _final_body,
        out_shape=jax.ShapeDtypeStruct((NP, C), jnp.float32),
        in_specs=[
            pl.BlockSpec(memory_space=pltpu.VMEM),
            pl.BlockSpec(memory_space=pltpu.VMEM),
            pl.BlockSpec(memory_space=pltpu.VMEM),
        ],
    )(u, den, W)


def _sc_layer_kernel(hn_hbm, hb_hbm, nrm_hbm, src_hbm, dst_hbm,
                     u_out, den_out,
                     nrm_v, sidx, didx, srows, drows, w_v,
                     zrow, zden, u_sp, den_sp, sem1, sem2):
    c = lax.axis_index("c")
    s = lax.axis_index("s")
    wid = c * 16 + s
    wbase = wid * EPW

    iota = lax.iota(jnp.int32, 16)
    zero16 = jnp.zeros((16,), jnp.float32)

    pltpu.sync_copy(nrm_hbm, nrm_v)

    for r in range(8):
        for k in range(8):
            zrow[r, pl.ds(k * 16, 16)] = zero16
    for k in range(RPT // 16):
        zden[pl.ds(k * 16, 16)] = zero16

    def zero_body(j, _):
        pltpu.sync_copy(zrow, u_sp.at[pl.ds(s * RPT + j * 8, 8)])
        return 0
    lax.fori_loop(0, RPT // 8, zero_body, 0)
    pltpu.sync_copy(zden, den_sp.at[pl.ds(s * RPT, RPT)])

    plsc.subcore_barrier()

    def chunk_body(i, _):
        base = wbase + i * CB
        pltpu.sync_copy(src_hbm.at[pl.ds(base, CB)], sidx)
        pltpu.sync_copy(dst_hbm.at[pl.ds(base, CB)], didx)
        cp1 = pltpu.async_copy(hn_hbm.at[sidx], srows, sem1)
        cp2 = pltpu.async_copy(hb_hbm.at[didx], drows, sem2)
        cp1.wait()
        cp2.wait()
        for g in range(CB // 16):
            rowv = iota + (g * 16)
            sidx16 = sidx[pl.ds(g * 16, 16)]

            def dot_body(kk, acc):
                for dk in range(8):
                    colv = jnp.full((16,), kk * 8 + dk, jnp.int32)
                    a = plsc.load_gather(srows, [rowv, colv])
                    b = plsc.load_gather(drows, [rowv, colv])
                    acc = acc + a * b
                return acc
            acc = lax.fori_loop(0, D // 8, dot_body, zero16)

            w = jnp.exp(acc)
            f = w * plsc.load_gather(nrm_v, [sidx16])
            w_v[pl.ds(g * 16, 16)] = w

            def scale_body(kk, t):
                for dk in range(8):
                    colv = jnp.full((16,), kk * 8 + dk, jnp.int32)
                    a = plsc.load_gather(srows, [rowv, colv])
                    plsc.store_scatter(srows, [rowv, colv], a * f)
                return t
            lax.fori_loop(0, D // 8, scale_body, 0)

        pltpu.sync_copy(srows, u_sp.at[didx], add=True)
        pltpu.sync_copy(w_v, den_sp.at[didx], add=True)
        return 0

    lax.fori_loop(0, NCH, chunk_body, 0)

    plsc.subcore_barrier()

    pltpu.sync_copy(u_sp.at[pl.ds(s * RPT, RPT)],
                    u_out.at[c, pl.ds(s * RPT, RPT)])
    pltpu.sync_copy(den_sp.at[pl.ds(s * RPT, RPT)],
                    den_out.at[c, pl.ds(s * RPT, RPT)])


@functools.partial(
    pl.kernel,
    mesh=plsc.VectorSubcoreMesh(core_axis_name="c", subcore_axis_name="s"),
    compiler_params=pltpu.CompilerParams(needs_layout_passes=False),
    out_type=[
        jax.ShapeDtypeStruct((2, NP, D), jnp.float32),
        jax.ShapeDtypeStruct((2, NP), jnp.float32),
    ],
    scratch_types=[
        pltpu.VMEM((NP,), jnp.float32),
        pltpu.VMEM((CB,), jnp.int32),
        pltpu.VMEM((CB,), jnp.int32),
        pltpu.VMEM((CB, D), jnp.float32),
        pltpu.VMEM((CB, D), jnp.float32),
        pltpu.VMEM((CB,), jnp.float32),
        pltpu.VMEM((8, D), jnp.float32),
        pltpu.VMEM((RPT,), jnp.float32),
        pltpu.VMEM_SHARED((NP, D), jnp.float32),
        pltpu.VMEM_SHARED((NP,), jnp.float32),
        pltpu.SemaphoreType.DMA,
        pltpu.SemaphoreType.DMA,
    ],
)
def _sc_layer(hn_hbm, hb_hbm, nrm_hbm, src_hbm, dst_hbm, u_out, den_out,
              *rest):
    _sc_layer_kernel(hn_hbm, hb_hbm, nrm_hbm, src_hbm, dst_hbm,
                     u_out, den_out, *rest)


def kernel(features, edge_index, betas, W):
    src = edge_index[0].astype(jnp.int32)
    dst = edge_index[1].astype(jnp.int32)
    xpad = jnp.pad(features, ((0, NP - N), (0, 0)))
    betas = betas.astype(jnp.float32)

    hn, hb, nrm = _prep(xpad, betas[0:1])
    for i in range(3):
        u, den = _sc_layer(hn, hb, nrm, src, dst)
        if i < 2:
            hn, hb, nrm = _combine(u, den, betas[i + 1:i + 2])
        else:
            y = _final(u, den, W)
    return y[:N]


# speedup vs baseline: 6.9870x; 3.5201x over previous
import functools

import jax
import jax.numpy as jnp
from jax import lax
from jax.experimental import pallas as pl
from jax.experimental.pallas import tpu as pltpu
from jax.experimental.pallas import tpu_sc as plsc

N = 10000
NP = 10240
D = 128
E = 320000
C = 64
NWORK = 32
EPW = E // NWORK
CB = 80
NCH = EPW // CB
RPT = NP // 16
EPS = 1e-12


def _prep_body(x_ref, b_ref, hn_ref, hb_ref, nrm_ref):
    x = x_ref[...]
    nrm = jnp.sqrt(jnp.sum(x * x, axis=1, keepdims=True))
    hn = x / jnp.maximum(nrm, EPS)
    hn_ref[...] = hn
    hb_ref[...] = hn * b_ref[0]
    nrm_ref[...] = nrm[:, 0]


def _prep(xpad, beta):
    return pl.pallas_call(
        _prep_body,
        out_shape=[
            jax.ShapeDtypeStruct((NP, D), jnp.float32),
            jax.ShapeDtypeStruct((NP, D), jnp.float32),
            jax.ShapeDtypeStruct((NP,), jnp.float32),
        ],
        in_specs=[
            pl.BlockSpec(memory_space=pltpu.VMEM),
            pl.BlockSpec(memory_space=pltpu.SMEM),
        ],
    )(xpad, beta)


def _combine_body(u_ref, den_ref, b_ref, hn_ref, hb_ref, nrm_ref):
    u = u_ref[0] + u_ref[1]
    den = den_ref[0] + den_ref[1]
    den = jnp.where(den == 0.0, 1.0, den)
    x = u / den[:, None]
    nrm = jnp.sqrt(jnp.sum(x * x, axis=1, keepdims=True))
    hn = x / jnp.maximum(nrm, EPS)
    hn_ref[...] = hn
    hb_ref[...] = hn * b_ref[0]
    nrm_ref[...] = nrm[:, 0]


def _combine(u, den, beta):
    return pl.pallas_call(
        _combine_body,
        out_shape=[
            jax.ShapeDtypeStruct((NP, D), jnp.float32),
            jax.ShapeDtypeStruct((NP, D), jnp.float32),
            jax.ShapeDtypeStruct((NP,), jnp.float32),
        ],
        in_specs=[
            pl.BlockSpec(memory_space=pltpu.VMEM),
            pl.BlockSpec(memory_space=pltpu.VMEM),
            pl.BlockSpec(memory_space=pltpu.SMEM),
        ],
    )(u, den, beta)


def _final_body(u_ref, den_ref, w_ref, out_ref):
    u = u_ref[0] + u_ref[1]
    den = den_ref[0] + den_ref[1]
    den = jnp.where(den == 0.0, 1.0, den)
    x = u / den[:, None]
    out_ref[...] = lax.dot_general(
        x, w_ref[...], (((1,), (1,)), ((), ())),
        preferred_element_type=jnp.float32)


def _final(u, den, W):
    return pl.pallas_call(
        _final_body,
        out_shape=jax.ShapeDtypeStruct((NP, C), jnp.float32),
        in_specs=[
            pl.BlockSpec(memory_space=pltpu.VMEM),
            pl.BlockSpec(memory_space=pltpu.VMEM),
            pl.BlockSpec(memory_space=pltpu.VMEM),
        ],
    )(u, den, W)


def _sc_layer_kernel(hn_hbm, hb_hbm, nrm_hbm, src_hbm, dst_hbm,
                     u_out, den_out,
                     nrm_v, sidx, didx, srows, drows, w_v,
                     zrow, zden, u_sp, den_sp, sem1, sem2):
    c = lax.axis_index("c")
    s = lax.axis_index("s")
    wid = c * 16 + s
    wbase = wid * EPW

    iota = lax.iota(jnp.int32, 16)
    zero16 = jnp.zeros((16,), jnp.float32)

    pltpu.sync_copy(nrm_hbm, nrm_v)

    for r in range(8):
        for k in range(8):
            zrow[r, pl.ds(k * 16, 16)] = zero16
    for k in range(RPT // 16):
        zden[pl.ds(k * 16, 16)] = zero16

    def zero_body(j, _):
        pltpu.sync_copy(zrow, u_sp.at[pl.ds(s * RPT + j * 8, 8)])
        return 0
    lax.fori_loop(0, RPT // 8, zero_body, 0)
    pltpu.sync_copy(zden, den_sp.at[pl.ds(s * RPT, RPT)])

    plsc.subcore_barrier()

    def chunk_body(i, _):
        base = wbase + i * CB
        pltpu.sync_copy(src_hbm.at[pl.ds(base, CB)], sidx)
        pltpu.sync_copy(dst_hbm.at[pl.ds(base, CB)], didx)
        cp1 = pltpu.async_copy(hn_hbm.at[sidx], srows, sem1)
        cp2 = pltpu.async_copy(hb_hbm.at[didx], drows, sem2)
        cp1.wait()
        cp2.wait()
        for g in range(CB // 16):
            rowv = iota + (g * 16)
            sidx16 = sidx[pl.ds(g * 16, 16)]

            def dot_body(kk, acc):
                for dk in range(8):
                    colv = (jnp.full((16,), kk * 8 + dk, jnp.int32) + iota) & (D - 1)
                    a = plsc.load_gather(srows, [rowv, colv])
                    b = plsc.load_gather(drows, [rowv, colv])
                    acc = acc + a * b
                return acc
            acc = lax.fori_loop(0, D // 8, dot_body, zero16)

            w = jnp.exp(acc)
            f = w * plsc.load_gather(nrm_v, [sidx16])
            w_v[pl.ds(g * 16, 16)] = w

            def scale_body(kk, t):
                for dk in range(8):
                    colv = (jnp.full((16,), kk * 8 + dk, jnp.int32) + iota) & (D - 1)
                    a = plsc.load_gather(srows, [rowv, colv])
                    plsc.store_scatter(srows, [rowv, colv], a * f)
                return t
            lax.fori_loop(0, D // 8, scale_body, 0)

        pltpu.sync_copy(srows, u_sp.at[didx], add=True)
        pltpu.sync_copy(w_v, den_sp.at[didx], add=True)
        return 0

    lax.fori_loop(0, NCH, chunk_body, 0)

    plsc.subcore_barrier()

    pltpu.sync_copy(u_sp.at[pl.ds(s * RPT, RPT)],
                    u_out.at[c, pl.ds(s * RPT, RPT)])
    pltpu.sync_copy(den_sp.at[pl.ds(s * RPT, RPT)],
                    den_out.at[c, pl.ds(s * RPT, RPT)])


@functools.partial(
    pl.kernel,
    mesh=plsc.VectorSubcoreMesh(core_axis_name="c", subcore_axis_name="s"),
    compiler_params=pltpu.CompilerParams(needs_layout_passes=False),
    out_type=[
        jax.ShapeDtypeStruct((2, NP, D), jnp.float32),
        jax.ShapeDtypeStruct((2, NP), jnp.float32),
    ],
    scratch_types=[
        pltpu.VMEM((NP,), jnp.float32),
        pltpu.VMEM((CB,), jnp.int32),
        pltpu.VMEM((CB,), jnp.int32),
        pltpu.VMEM((CB, D), jnp.float32),
        pltpu.VMEM((CB, D), jnp.float32),
        pltpu.VMEM((CB,), jnp.float32),
        pltpu.VMEM((8, D), jnp.float32),
        pltpu.VMEM((RPT,), jnp.float32),
        pltpu.VMEM_SHARED((NP, D), jnp.float32),
        pltpu.VMEM_SHARED((NP,), jnp.float32),
        pltpu.SemaphoreType.DMA,
        pltpu.SemaphoreType.DMA,
    ],
)
def _sc_layer(hn_hbm, hb_hbm, nrm_hbm, src_hbm, dst_hbm, u_out, den_out,
              *rest):
    _sc_layer_kernel(hn_hbm, hb_hbm, nrm_hbm, src_hbm, dst_hbm,
                     u_out, den_out, *rest)


def kernel(features, edge_index, betas, W):
    src = edge_index[0].astype(jnp.int32)
    dst = edge_index[1].astype(jnp.int32)
    xpad = jnp.pad(features, ((0, NP - N), (0, 0)))
    betas = betas.astype(jnp.float32)

    hn, hb, nrm = _prep(xpad, betas[0:1])
    for i in range(3):
        u, den = _sc_layer(hn, hb, nrm, src, dst)
        if i < 2:
            hn, hb, nrm = _combine(u, den, betas[i + 1:i + 2])
        else:
            y = _final(u, den, W)
    return y[:N]


# speedup vs baseline: 7.2069x; 1.0315x over previous
import functools

import jax
import jax.numpy as jnp
from jax import lax
from jax.experimental import pallas as pl
from jax.experimental.pallas import tpu as pltpu
from jax.experimental.pallas import tpu_sc as plsc

N = 10000
NP = 10240
NA = 10112
D = 128
E = 320000
C = 64
NWORK = 32
EPW = E // NWORK
CB = 80
NCH = EPW // CB
NG = CB // 16
RPT_U = NA // 16
RPT_D = NP // 16
EPS = 1e-12


def _prep_body(x_ref, b_ref, hn_ref, hb_ref, nrm_ref):
    x = x_ref[...]
    nrm = jnp.sqrt(jnp.sum(x * x, axis=1, keepdims=True))
    hn = x / jnp.maximum(nrm, EPS)
    hn_ref[...] = hn
    hb_ref[...] = hn * b_ref[0]
    nrm_ref[...] = nrm[:, 0]


def _prep(xpad, beta):
    return pl.pallas_call(
        _prep_body,
        out_shape=[
            jax.ShapeDtypeStruct((NP, D), jnp.float32),
            jax.ShapeDtypeStruct((NP, D), jnp.float32),
            jax.ShapeDtypeStruct((NP,), jnp.float32),
        ],
        in_specs=[
            pl.BlockSpec(memory_space=pltpu.VMEM),
            pl.BlockSpec(memory_space=pltpu.SMEM),
        ],
    )(xpad, beta)


def _combine_body(u_ref, den_ref, b_ref, hn_ref, hb_ref, nrm_ref):
    u = u_ref[0] + u_ref[1]
    den = den_ref[0, pl.ds(0, NA)] + den_ref[1, pl.ds(0, NA)]
    den = jnp.where(den == 0.0, 1.0, den)
    x = u / den[:, None]
    nrm = jnp.sqrt(jnp.sum(x * x, axis=1, keepdims=True))
    hn = x / jnp.maximum(nrm, EPS)
    pad = jnp.zeros((NP - NA, D), jnp.float32)
    hn_full = jnp.concatenate([hn, pad], axis=0)
    hn_ref[...] = hn_full
    hb_ref[...] = hn_full * b_ref[0]
    nrm_ref[...] = jnp.concatenate([nrm[:, 0], jnp.zeros((NP - NA,), jnp.float32)])


def _combine(u, den, beta):
    return pl.pallas_call(
        _combine_body,
        out_shape=[
            jax.ShapeDtypeStruct((NP, D), jnp.float32),
            jax.ShapeDtypeStruct((NP, D), jnp.float32),
            jax.ShapeDtypeStruct((NP,), jnp.float32),
        ],
        in_specs=[
            pl.BlockSpec(memory_space=pltpu.VMEM),
            pl.BlockSpec(memory_space=pltpu.VMEM),
            pl.BlockSpec(memory_space=pltpu.SMEM),
        ],
    )(u, den, beta)


def _final_body(u_ref, den_ref, w_ref, out_ref):
    u = u_ref[0] + u_ref[1]
    den = den_ref[0, pl.ds(0, NA)] + den_ref[1, pl.ds(0, NA)]
    den = jnp.where(den == 0.0, 1.0, den)
    x = u / den[:, None]
    out_ref[...] = lax.dot_general(
        x, w_ref[...], (((1,), (1,)), ((), ())),
        preferred_element_type=jnp.float32)


def _final(u, den, W):
    return pl.pallas_call(
        _final_body,
        out_shape=jax.ShapeDtypeStruct((NA, C), jnp.float32),
        in_specs=[
            pl.BlockSpec(memory_space=pltpu.VMEM),
            pl.BlockSpec(memory_space=pltpu.VMEM),
            pl.BlockSpec(memory_space=pltpu.VMEM),
        ],
    )(u, den, W)


def _sc_layer_kernel(hn_hbm, hb_hbm, nrm_hbm, src_hbm, dst_hbm,
                     u_out, den_out,
                     sia, dia, sib, dib,
                     srows_a, drows_a, nrm_a, w_a,
                     srows_b, drows_b, nrm_b, w_b,
                     zrow, zden, u_sp, den_sp, sem_a, sem_b):
    c = lax.axis_index("c")
    s = lax.axis_index("s")
    wid = c * 16 + s

    iota = lax.iota(jnp.int32, 16)
    zero16 = jnp.zeros((16,), jnp.float32)

    for r in range(4):
        for k in range(8):
            zrow[r, pl.ds(k * 16, 16)] = zero16
    for k in range(RPT_D // 16):
        zden[pl.ds(k * 16, 16)] = zero16

    def zero_body(j, _):
        pltpu.sync_copy(zrow, u_sp.at[pl.ds(s * RPT_U + j * 4, 4)])
        return 0
    lax.fori_loop(0, RPT_U // 4, zero_body, 0)
    pltpu.sync_copy(zden, den_sp.at[pl.ds(s * RPT_D, RPT_D)])

    plsc.subcore_barrier()

    wbase = wid * EPW

    def load_idx(i, si, di):
        pltpu.sync_copy(src_hbm.at[pl.ds(wbase + i * CB, CB)], si)
        pltpu.sync_copy(dst_hbm.at[pl.ds(wbase + i * CB, CB)], di)

    def fire(si, di, rows_s, rows_d, nrm_g, sem):
        pltpu.async_copy(hn_hbm.at[si], rows_s, sem)
        pltpu.async_copy(hb_hbm.at[di], rows_d, sem)
        pltpu.async_copy(nrm_hbm.at[si], nrm_g, sem)

    def wait(si, di, rows_s, rows_d, nrm_g, sem):
        pltpu.make_async_copy(hn_hbm.at[si], rows_s, sem).wait()
        pltpu.make_async_copy(hb_hbm.at[di], rows_d, sem).wait()
        pltpu.make_async_copy(nrm_hbm.at[si], nrm_g, sem).wait()

    def compute_and_scatter(srows, drows, nrm_g, w_v, di):
        for g in range(NG):
            rowv = iota + (g * 16)

            def dot_body(kk, acc):
                for dk in range(16):
                    colv = (jnp.full((16,), kk * 16 + dk, jnp.int32)
                            + iota) & (D - 1)
                    a = plsc.load_gather(srows, [rowv, colv])
                    b = plsc.load_gather(drows, [rowv, colv])
                    acc = acc + a * b
                return acc
            acc = lax.fori_loop(0, D // 16, dot_body, zero16)

            w = jnp.exp(acc)
            f = w * nrm_g[pl.ds(g * 16, 16)]
            w_v[pl.ds(g * 16, 16)] = w

            def scale_body(kk, t):
                for dk in range(16):
                    colv = (jnp.full((16,), kk * 16 + dk, jnp.int32)
                            + iota) & (D - 1)
                    a = plsc.load_gather(srows, [rowv, colv])
                    plsc.store_scatter(srows, [rowv, colv], a * f)
                return t
            lax.fori_loop(0, D // 16, scale_body, 0)

        pltpu.sync_copy(srows, u_sp.at[di], add=True)
        pltpu.sync_copy(w_v, den_sp.at[di], add=True)

    load_idx(0, sia, dia)
    fire(sia, dia, srows_a, drows_a, nrm_a, sem_a)

    def pair_body(t, _):
        i0 = 2 * t
        load_idx(i0 + 1, sib, dib)
        fire(sib, dib, srows_b, drows_b, nrm_b, sem_b)
        wait(sia, dia, srows_a, drows_a, nrm_a, sem_a)
        compute_and_scatter(srows_a, drows_a, nrm_a, w_a, dia)
        load_idx(i0 + 2, sia, dia)
        fire(sia, dia, srows_a, drows_a, nrm_a, sem_a)
        wait(sib, dib, srows_b, drows_b, nrm_b, sem_b)
        compute_and_scatter(srows_b, drows_b, nrm_b, w_b, dib)
        return 0

    lax.fori_loop(0, (NCH - 1) // 2, pair_body, 0)

    wait(sia, dia, srows_a, drows_a, nrm_a, sem_a)
    compute_and_scatter(srows_a, drows_a, nrm_a, w_a, dia)

    plsc.subcore_barrier()

    pltpu.sync_copy(u_sp.at[pl.ds(s * RPT_U, RPT_U)],
                    u_out.at[c, pl.ds(s * RPT_U, RPT_U)])
    pltpu.sync_copy(den_sp.at[pl.ds(s * RPT_D, RPT_D)],
                    den_out.at[c, pl.ds(s * RPT_D, RPT_D)])


@functools.partial(
    pl.kernel,
    mesh=plsc.VectorSubcoreMesh(core_axis_name="c", subcore_axis_name="s"),
    compiler_params=pltpu.CompilerParams(needs_layout_passes=False),
    out_type=[
        jax.ShapeDtypeStruct((2, NA, D), jnp.float32),
        jax.ShapeDtypeStruct((2, NP), jnp.float32),
    ],
    scratch_types=[
        pltpu.VMEM((CB,), jnp.int32),
        pltpu.VMEM((CB,), jnp.int32),
        pltpu.VMEM((CB,), jnp.int32),
        pltpu.VMEM((CB,), jnp.int32),
        pltpu.VMEM((CB, D), jnp.float32),
        pltpu.VMEM((CB, D), jnp.float32),
        pltpu.VMEM((CB,), jnp.float32),
        pltpu.VMEM((CB,), jnp.float32),
        pltpu.VMEM((CB, D), jnp.float32),
        pltpu.VMEM((CB, D), jnp.float32),
        pltpu.VMEM((CB,), jnp.float32),
        pltpu.VMEM((CB,), jnp.float32),
        pltpu.VMEM((4, D), jnp.float32),
        pltpu.VMEM((RPT_D,), jnp.float32),
        pltpu.VMEM_SHARED((NA, D), jnp.float32),
        pltpu.VMEM_SHARED((NP,), jnp.float32),
        pltpu.SemaphoreType.DMA,
        pltpu.SemaphoreType.DMA,
    ],
)
def _sc_layer(hn_hbm, hb_hbm, nrm_hbm, src_hbm, dst_hbm, u_out, den_out,
              *rest):
    _sc_layer_kernel(hn_hbm, hb_hbm, nrm_hbm, src_hbm, dst_hbm,
                     u_out, den_out, *rest)


def kernel(features, edge_index, betas, W):
    src = edge_index[0].astype(jnp.int32)
    dst = edge_index[1].astype(jnp.int32)
    xpad = jnp.pad(features, ((0, NP - N), (0, 0)))
    betas = betas.astype(jnp.float32)

    hn, hb, nrm = _prep(xpad, betas[0:1])
    for i in range(3):
        u, den = _sc_layer(hn, hb, nrm, src, dst)
        if i < 2:
            hn, hb, nrm = _combine(u, den, betas[i + 1:i + 2])
        else:
            y = _final(u, den, W)
    return y[:N]


# speedup vs baseline: 7.6288x; 1.0585x over previous
import functools

import jax
import jax.numpy as jnp
from jax import lax
from jax.experimental import pallas as pl
from jax.experimental.pallas import tpu as pltpu
from jax.experimental.pallas import tpu_sc as plsc

N = 10000
NP = 10240
NA = 10112
D = 128
E = 320000
C = 64
NWORK = 32
EPW = E // NWORK
CB = 80
NCH = EPW // CB
NG = CB // 16
RPT_U = NA // 16
RPT_D = NP // 16
EPS = 1e-12


def _prep_body(x_ref, b_ref, hn_ref, hb_ref, nrm_ref):
    x = x_ref[...]
    nrm = jnp.sqrt(jnp.sum(x * x, axis=1, keepdims=True))
    hn = x / jnp.maximum(nrm, EPS)
    hn_ref[...] = hn
    hb_ref[...] = hn * b_ref[0]
    nrm_ref[...] = nrm[:, 0]


def _prep(xpad, beta):
    return pl.pallas_call(
        _prep_body,
        out_shape=[
            jax.ShapeDtypeStruct((NP, D), jnp.float32),
            jax.ShapeDtypeStruct((NP, D), jnp.float32),
            jax.ShapeDtypeStruct((NP,), jnp.float32),
        ],
        in_specs=[
            pl.BlockSpec(memory_space=pltpu.VMEM),
            pl.BlockSpec(memory_space=pltpu.SMEM),
        ],
    )(xpad, beta)


def _combine_body(u_ref, den_ref, b_ref, hn_ref, hb_ref, nrm_ref):
    u = u_ref[0] + u_ref[1]
    den = den_ref[0, pl.ds(0, NA)] + den_ref[1, pl.ds(0, NA)]
    den = jnp.where(den == 0.0, 1.0, den)
    x = u / den[:, None]
    nrm = jnp.sqrt(jnp.sum(x * x, axis=1, keepdims=True))
    hn = x / jnp.maximum(nrm, EPS)
    pad = jnp.zeros((NP - NA, D), jnp.float32)
    hn_full = jnp.concatenate([hn, pad], axis=0)
    hn_ref[...] = hn_full
    hb_ref[...] = hn_full * b_ref[0]
    nrm_ref[...] = jnp.concatenate([nrm[:, 0], jnp.zeros((NP - NA,), jnp.float32)])


def _combine(u, den, beta):
    return pl.pallas_call(
        _combine_body,
        out_shape=[
            jax.ShapeDtypeStruct((NP, D), jnp.float32),
            jax.ShapeDtypeStruct((NP, D), jnp.float32),
            jax.ShapeDtypeStruct((NP,), jnp.float32),
        ],
        in_specs=[
            pl.BlockSpec(memory_space=pltpu.VMEM),
            pl.BlockSpec(memory_space=pltpu.VMEM),
            pl.BlockSpec(memory_space=pltpu.SMEM),
        ],
    )(u, den, beta)


def _final_body(u_ref, den_ref, w_ref, out_ref):
    u = u_ref[0] + u_ref[1]
    den = den_ref[0, pl.ds(0, NA)] + den_ref[1, pl.ds(0, NA)]
    den = jnp.where(den == 0.0, 1.0, den)
    x = u / den[:, None]
    out_ref[...] = lax.dot_general(
        x, w_ref[...], (((1,), (1,)), ((), ())),
        preferred_element_type=jnp.float32)


def _final(u, den, W):
    return pl.pallas_call(
        _final_body,
        out_shape=jax.ShapeDtypeStruct((NA, C), jnp.float32),
        in_specs=[
            pl.BlockSpec(memory_space=pltpu.VMEM),
            pl.BlockSpec(memory_space=pltpu.VMEM),
            pl.BlockSpec(memory_space=pltpu.VMEM),
        ],
    )(u, den, W)


def _sc_layer_kernel(hn_hbm, hb_hbm, nrm_hbm, src_hbm, dst_hbm,
                     u_out, den_out,
                     sia, dia, sib, dib,
                     srows_a, drows_a, nrm_a, w_a,
                     srows_b, drows_b, nrm_b, w_b,
                     zrow, zden, u_sp, den_sp, sem_a, sem_b):
    c = lax.axis_index("c")
    s = lax.axis_index("s")
    wid = c * 16 + s

    iota = lax.iota(jnp.int32, 16)
    zero16 = jnp.zeros((16,), jnp.float32)

    for r in range(4):
        for k in range(8):
            zrow[r, pl.ds(k * 16, 16)] = zero16
    for k in range(RPT_D // 16):
        zden[pl.ds(k * 16, 16)] = zero16

    def zero_body(j, _):
        pltpu.sync_copy(zrow, u_sp.at[pl.ds(s * RPT_U + j * 4, 4)])
        return 0
    lax.fori_loop(0, RPT_U // 4, zero_body, 0)
    pltpu.sync_copy(zden, den_sp.at[pl.ds(s * RPT_D, RPT_D)])

    plsc.subcore_barrier()

    wbase = wid * EPW

    def load_idx(i, si, di):
        pltpu.sync_copy(src_hbm.at[pl.ds(wbase + i * CB, CB)], si)
        pltpu.sync_copy(dst_hbm.at[pl.ds(wbase + i * CB, CB)], di)

    def fire(si, di, rows_s, rows_d, nrm_g, sem):
        pltpu.async_copy(hn_hbm.at[si], rows_s, sem)
        pltpu.async_copy(hb_hbm.at[di], rows_d, sem)
        pltpu.async_copy(nrm_hbm.at[si], nrm_g, sem)

    def wait(si, di, rows_s, rows_d, nrm_g, sem):
        pltpu.make_async_copy(hn_hbm.at[si], rows_s, sem).wait()
        pltpu.make_async_copy(hb_hbm.at[di], rows_d, sem).wait()
        pltpu.make_async_copy(nrm_hbm.at[si], nrm_g, sem).wait()

    def compute_and_scatter(srows, drows, nrm_g, w_v, di):
        for g in range(NG):
            rowv = iota + (g * 16)

            def dot_body(kk, accs):
                accs = list(accs)
                for dk in range(16):
                    colv = (jnp.full((16,), kk * 16 + dk, jnp.int32)
                            + iota) & (D - 1)
                    a = plsc.load_gather(srows, [rowv, colv])
                    b = plsc.load_gather(drows, [rowv, colv])
                    accs[dk % 4] = accs[dk % 4] + a * b
                return tuple(accs)
            a0, a1, a2, a3 = lax.fori_loop(
                0, D // 16, dot_body, (zero16, zero16, zero16, zero16))
            acc = (a0 + a1) + (a2 + a3)

            w = jnp.exp(acc)
            f = w * nrm_g[pl.ds(g * 16, 16)]
            w_v[pl.ds(g * 16, 16)] = w

            def scale_body(kk, t):
                for dk in range(16):
                    colv = (jnp.full((16,), kk * 16 + dk, jnp.int32)
                            + iota) & (D - 1)
                    a = plsc.load_gather(srows, [rowv, colv])
                    plsc.store_scatter(srows, [rowv, colv], a * f)
                return t
            lax.fori_loop(0, D // 16, scale_body, 0)

        pltpu.sync_copy(srows, u_sp.at[di], add=True)
        pltpu.sync_copy(w_v, den_sp.at[di], add=True)

    load_idx(0, sia, dia)
    fire(sia, dia, srows_a, drows_a, nrm_a, sem_a)

    def pair_body(t, _):
        i0 = 2 * t
        load_idx(i0 + 1, sib, dib)
        fire(sib, dib, srows_b, drows_b, nrm_b, sem_b)
        wait(sia, dia, srows_a, drows_a, nrm_a, sem_a)
        compute_and_scatter(srows_a, drows_a, nrm_a, w_a, dia)
        load_idx(i0 + 2, sia, dia)
        fire(sia, dia, srows_a, drows_a, nrm_a, sem_a)
        wait(sib, dib, srows_b, drows_b, nrm_b, sem_b)
        compute_and_scatter(srows_b, drows_b, nrm_b, w_b, dib)
        return 0

    lax.fori_loop(0, (NCH - 1) // 2, pair_body, 0)

    wait(sia, dia, srows_a, drows_a, nrm_a, sem_a)
    compute_and_scatter(srows_a, drows_a, nrm_a, w_a, dia)

    plsc.subcore_barrier()

    pltpu.sync_copy(u_sp.at[pl.ds(s * RPT_U, RPT_U)],
                    u_out.at[c, pl.ds(s * RPT_U, RPT_U)])
    pltpu.sync_copy(den_sp.at[pl.ds(s * RPT_D, RPT_D)],
                    den_out.at[c, pl.ds(s * RPT_D, RPT_D)])


@functools.partial(
    pl.kernel,
    mesh=plsc.VectorSubcoreMesh(core_axis_name="c", subcore_axis_name="s"),
    compiler_params=pltpu.CompilerParams(needs_layout_passes=False),
    out_type=[
        jax.ShapeDtypeStruct((2, NA, D), jnp.float32),
        jax.ShapeDtypeStruct((2, NP), jnp.float32),
    ],
    scratch_types=[
        pltpu.VMEM((CB,), jnp.int32),
        pltpu.VMEM((CB,), jnp.int32),
        pltpu.VMEM((CB,), jnp.int32),
        pltpu.VMEM((CB,), jnp.int32),
        pltpu.VMEM((CB, D), jnp.float32),
        pltpu.VMEM((CB, D), jnp.float32),
        pltpu.VMEM((CB,), jnp.float32),
        pltpu.VMEM((CB,), jnp.float32),
        pltpu.VMEM((CB, D), jnp.float32),
        pltpu.VMEM((CB, D), jnp.float32),
        pltpu.VMEM((CB,), jnp.float32),
        pltpu.VMEM((CB,), jnp.float32),
        pltpu.VMEM((4, D), jnp.float32),
        pltpu.VMEM((RPT_D,), jnp.float32),
        pltpu.VMEM_SHARED((NA, D), jnp.float32),
        pltpu.VMEM_SHARED((NP,), jnp.float32),
        pltpu.SemaphoreType.DMA,
        pltpu.SemaphoreType.DMA,
    ],
)
def _sc_layer(hn_hbm, hb_hbm, nrm_hbm, src_hbm, dst_hbm, u_out, den_out,
              *rest):
    _sc_layer_kernel(hn_hbm, hb_hbm, nrm_hbm, src_hbm, dst_hbm,
                     u_out, den_out, *rest)


def kernel(features, edge_index, betas, W):
    src = edge_index[0].astype(jnp.int32)
    dst = edge_index[1].astype(jnp.int32)
    xpad = jnp.pad(features, ((0, NP - N), (0, 0)))
    betas = betas.astype(jnp.float32)

    hn, hb, nrm = _prep(xpad, betas[0:1])
    for i in range(3):
        u, den = _sc_layer(hn, hb, nrm, src, dst)
        if i < 2:
            hn, hb, nrm = _combine(u, den, betas[i + 1:i + 2])
        else:
            y = _final(u, den, W)
    return y[:N]
